# baseline (speedup 1.0000x reference)
import functools

import jax
import jax.numpy as jnp
from jax import lax
from jax.experimental import pallas as pl
from jax.experimental.pallas import tpu as pltpu
from jax.experimental.pallas import tpu_sc as plsc

D_MODEL = 1024
SCALE = float(D_MODEL) ** 0.5
L = 16
NC = 2
NS = 16
NW = NC * NS
CH = 32
BATCH = 4


@functools.lru_cache(maxsize=None)
def _make_sc_kernel(V, D, BF, S):
  R = BF // NW
  NCH = R // CH
  SR = CH // BATCH
  mesh = plsc.VectorSubcoreMesh(core_axis_name="c", subcore_axis_name="s")

  @functools.partial(
      pl.kernel,
      mesh=mesh,
      out_type=jax.ShapeDtypeStruct((BF, D), jnp.float32),
      scratch_types=[
          pltpu.VMEM((R,), jnp.int32),
          pltpu.VMEM((CH, D), jnp.float32),
          pltpu.VMEM((SR, D), jnp.float32),
          pltpu.SemaphoreType.DMA,
      ],
  )
  def k(w_hbm, idx_hbm, pe_hbm, out_hbm, idx_v, rows_v, pe_v, sem):
    wid = lax.axis_index("s") * NC + lax.axis_index("c")
    base = pl.multiple_of(wid * R, R)
    pltpu.sync_copy(idx_hbm.at[pl.ds(base, R)], idx_v)
    for c in range(NCH):
      row0 = pl.multiple_of(base + c * CH, CH)
      pe_row0 = pl.multiple_of(row0 // BATCH, CH // BATCH)
      pltpu.async_copy(w_hbm.at[idx_v.at[pl.ds(c * CH, CH)]], rows_v, sem).wait()
      pltpu.sync_copy(pe_hbm.at[pl.ds(pe_row0, SR)], pe_v)

      def body(t, carry):
        s_loc = t // (D // L)
        col = (t % (D // L)) * L
        col = pl.multiple_of(col, L)
        pe_reg = pe_v[s_loc, pl.ds(col, L)]
        for b in range(BATCH):
          r = s_loc * BATCH + b
          rows_v[r, pl.ds(col, L)] = rows_v[r, pl.ds(col, L)] * SCALE + pe_reg
        return carry

      lax.fori_loop(0, SR * (D // L), body, 0)
      pltpu.sync_copy(rows_v, out_hbm.at[pl.ds(row0, CH)])

  return k


def kernel(x, W, pe):
  S, B = x.shape
  V, D = W.shape
  x_flat = x.reshape(S * B)
  pe2 = pe[:S, 0, :]
  out = _make_sc_kernel(V, D, S * B, S)(W, x_flat, pe2)
  return out.reshape(S, B, D)


# speedup vs baseline: 1.0190x; 1.0190x over previous
import functools

import jax
import jax.numpy as jnp
from jax import lax
from jax.experimental import pallas as pl
from jax.experimental.pallas import tpu as pltpu
from jax.experimental.pallas import tpu_sc as plsc

D_MODEL = 1024
SCALE = float(D_MODEL) ** 0.5
L = 16
NC = 2
NS = 16
NW = NC * NS
CH = 32
BATCH = 4
UJ = 8


@functools.lru_cache(maxsize=None)
def _make_sc_kernel(V, D, BF, S):
  R = BF // NW
  NCH = R // CH
  SR = CH // BATCH
  mesh = plsc.VectorSubcoreMesh(core_axis_name="c", subcore_axis_name="s")

  @functools.partial(
      pl.kernel,
      mesh=mesh,
      out_type=jax.ShapeDtypeStruct((BF, D), jnp.float32),
      scratch_types=[
          pltpu.VMEM((R,), jnp.int32),
          pltpu.VMEM((CH, D), jnp.float32),
          pltpu.VMEM((CH, D), jnp.float32),
          pltpu.VMEM((SR, D), jnp.float32),
          pltpu.VMEM((SR, D), jnp.float32),
          pltpu.SemaphoreType.DMA,
          pltpu.SemaphoreType.DMA,
          pltpu.SemaphoreType.DMA,
          pltpu.SemaphoreType.DMA,
      ],
  )
  def k(w_hbm, idx_hbm, pe_hbm, out_hbm,
        idx_v, rows_v0, rows_v1, pe_v0, pe_v1, gs0, gs1, ws0, ws1):
    wid = lax.axis_index("s") * NC + lax.axis_index("c")
    base = pl.multiple_of(wid * R, R)
    pltpu.sync_copy(idx_hbm.at[pl.ds(base, R)], idx_v)

    rows_b = (rows_v0, rows_v1)
    pe_b = (pe_v0, pe_v1)
    gs = (gs0, gs1)
    ws = (ws0, ws1)

    def start_load(c):
      buf = c % 2
      row0 = pl.multiple_of(base + c * CH, CH)
      pe_row0 = pl.multiple_of(row0 // BATCH, SR)
      g1 = pltpu.async_copy(
          w_hbm.at[idx_v.at[pl.ds(c * CH, CH)]], rows_b[buf], gs[buf])
      g2 = pltpu.async_copy(pe_hbm.at[pl.ds(pe_row0, SR)], pe_b[buf], gs[buf])
      return (g1, g2)

    def compute(rows_v, pe_v):
      def s_body(s_loc, carry):
        def j_body(jo, carry2):
          col0 = jo * (UJ * L)
          for ju in range(UJ):
            col = pl.multiple_of(col0 + ju * L, L)
            pe_reg = pe_v[s_loc, pl.ds(col, L)]
            for b in range(BATCH):
              r = s_loc * BATCH + b
              rows_v[r, pl.ds(col, L)] = rows_v[r, pl.ds(col, L)] * SCALE + pe_reg
          return carry2
        lax.fori_loop(0, (D // L) // UJ, j_body, 0)
        return carry
      lax.fori_loop(0, SR, s_body, 0)

    loads = [None] * NCH
    wbs = [None] * NCH
    loads[0] = start_load(0)
    for c in range(NCH):
      buf = c % 2
      row0 = pl.multiple_of(base + c * CH, CH)
      for g in loads[c]:
        g.wait()
      compute(rows_b[buf], pe_b[buf])
      wbs[c] = pltpu.async_copy(rows_b[buf], out_hbm.at[pl.ds(row0, CH)], ws[buf])
      if c + 1 < NCH:
        if c >= 1:
          wbs[c - 1].wait()
        loads[c + 1] = start_load(c + 1)
    wbs[NCH - 2].wait()
    wbs[NCH - 1].wait()

  return k


def kernel(x, W, pe):
  S, B = x.shape
  V, D = W.shape
  x_flat = x.reshape(S * B)
  pe2 = pe[:S, 0, :]
  out = _make_sc_kernel(V, D, S * B, S)(W, x_flat, pe2)
  return out.reshape(S, B, D)


# speedup vs baseline: 1.3460x; 1.3209x over previous
import functools

import jax
import jax.numpy as jnp
from jax import lax
from jax.experimental import pallas as pl
from jax.experimental.pallas import tpu as pltpu
from jax.experimental.pallas import tpu_sc as plsc

D_MODEL = 1024
SCALE = float(D_MODEL) ** 0.5
L = 16
NC = 2
NS = 16
NW = NC * NS
CH = 32
BATCH = 4
UJ = 8


@functools.lru_cache(maxsize=None)
def _make_sc_kernel(V, D, S, B, PEMAX):
  BF = S * B
  R = BF // NW
  NCH = R // CH
  SR = CH // B
  mesh = plsc.VectorSubcoreMesh(core_axis_name="c", subcore_axis_name="s")

  @functools.partial(
      pl.kernel,
      mesh=mesh,
      out_type=jax.ShapeDtypeStruct((S, B, D), jnp.float32),
      scratch_types=[
          pltpu.VMEM((R,), jnp.int32),
          pltpu.VMEM((CH, D), jnp.float32),
          pltpu.VMEM((CH, D), jnp.float32),
          pltpu.VMEM((SR, D), jnp.float32),
          pltpu.VMEM((SR, D), jnp.float32),
          pltpu.SemaphoreType.DMA,
          pltpu.SemaphoreType.DMA,
          pltpu.SemaphoreType.DMA,
          pltpu.SemaphoreType.DMA,
      ],
  )
  def k(w_hbm, idx_hbm, pe_hbm, out_hbm,
        idx_v, rows_v0, rows_v1, pe_v0, pe_v1, gs0, gs1, ws0, ws1):
    wid = lax.axis_index("s") * NC + lax.axis_index("c")
    base = pl.multiple_of(wid * R, R)
    sbase = pl.multiple_of(wid * (R // B), R // B)
    pltpu.sync_copy(idx_hbm.at[pl.ds(base, R)], idx_v)

    rows_b = (rows_v0, rows_v1)
    pe_b = (pe_v0, pe_v1)
    gs = (gs0, gs1)
    ws = (ws0, ws1)

    def start_load(c):
      buf = c % 2
      pe_row0 = pl.multiple_of(sbase + c * SR, SR)
      g1 = pltpu.async_copy(
          w_hbm.at[idx_v.at[pl.ds(c * CH, CH)]], rows_b[buf], gs[buf])
      g2 = pltpu.async_copy(pe_hbm.at[pl.ds(pe_row0, SR)], pe_b[buf], gs[buf])
      return (g1, g2)

    def compute(rows_v, pe_v):
      def s_body(s_loc, carry):
        def j_body(jo, carry2):
          col0 = jo * (UJ * L)
          for ju in range(UJ):
            col = pl.multiple_of(col0 + ju * L, L)
            pe_reg = pe_v[s_loc, pl.ds(col, L)]
            for b in range(B):
              r = s_loc * B + b
              rows_v[r, pl.ds(col, L)] = rows_v[r, pl.ds(col, L)] * SCALE + pe_reg
          return carry2
        lax.fori_loop(0, (D // L) // UJ, j_body, 0)
        return carry
      lax.fori_loop(0, SR, s_body, 0)

    def start_store(c):
      buf = c % 2
      hs = []
      for s in range(SR):
        hs.append(pltpu.async_copy(
            rows_b[buf].at[pl.ds(s * B, B)],
            out_hbm.at[sbase + c * SR + s], ws[buf]))
      return hs

    loads = [None] * NCH
    wbs = [None] * NCH
    loads[0] = start_load(0)
    for c in range(NCH):
      buf = c % 2
      for g in loads[c]:
        g.wait()
      compute(rows_b[buf], pe_b[buf])
      wbs[c] = start_store(c)
      if c + 1 < NCH:
        if c >= 1:
          for h in wbs[c - 1]:
            h.wait()
        loads[c + 1] = start_load(c + 1)
    for c in (NCH - 2, NCH - 1):
      for h in wbs[c]:
        h.wait()

  return k


def kernel(x, W, pe):
  S, B = x.shape
  V, D = W.shape
  x_flat = x.reshape(S * B)
  pe2 = pe.reshape(pe.shape[0], D)
  return _make_sc_kernel(V, D, S, B, pe.shape[0])(W, x_flat, pe2)


# speedup vs baseline: 2.0962x; 1.5574x over previous
import functools

import jax
import jax.numpy as jnp
from jax import lax
from jax.experimental import pallas as pl
from jax.experimental.pallas import tpu as pltpu
from jax.experimental.pallas import tpu_sc as plsc

D_MODEL = 1024
SCALE = float(D_MODEL) ** 0.5
L = 16
NC = 2
NS = 16
NW = NC * NS
CH = 32
UJ = 8


@functools.lru_cache(maxsize=None)
def _make_sc_kernel(V, D, S, B, PEMAX):
  BF = S * B
  R = BF // NW
  NCH = R // CH
  SR = CH // B
  mesh = plsc.VectorSubcoreMesh(core_axis_name="c", subcore_axis_name="s")

  @functools.partial(
      pl.kernel,
      mesh=mesh,
      out_type=jax.ShapeDtypeStruct((S, B, D), jnp.float32),
      scratch_types=[
          pltpu.VMEM((R,), jnp.int32),
          pltpu.VMEM((CH, D), jnp.float32),
          pltpu.VMEM((CH, D), jnp.float32),
          pltpu.VMEM((SR, D), jnp.float32),
          pltpu.VMEM((SR, D), jnp.float32),
          pltpu.SemaphoreType.DMA,
          pltpu.SemaphoreType.DMA,
          pltpu.SemaphoreType.DMA,
          pltpu.SemaphoreType.DMA,
      ],
  )
  def k(w_hbm, idx_hbm, pe_hbm, out_hbm,
        idx_v, rows_v0, rows_v1, pe_v0, pe_v1, gs0, gs1, ws0, ws1):
    pe2 = pe_hbm.reshape(PEMAX, D)
    out2 = out_hbm.reshape(BF, D)
    wid = lax.axis_index("s") * NC + lax.axis_index("c")
    base = pl.multiple_of(wid * R, R)
    pltpu.sync_copy(idx_hbm.at[pl.ds(base, R)], idx_v)

    rows_b = (rows_v0, rows_v1)
    pe_b = (pe_v0, pe_v1)
    gs = (gs0, gs1)
    ws = (ws0, ws1)

    def start_load(c):
      buf = c % 2
      pe_row0 = pl.multiple_of((base + c * CH) // B, SR)
      g1 = pltpu.async_copy(
          w_hbm.at[idx_v.at[pl.ds(c * CH, CH)]], rows_b[buf], gs[buf])
      g2 = pltpu.async_copy(pe2.at[pl.ds(pe_row0, SR)], pe_b[buf], gs[buf])
      return (g1, g2)

    def compute(rows_v, pe_v):
      def s_body(s_loc, carry):
        def j_body(jo, carry2):
          col0 = jo * (UJ * L)
          for ju in range(UJ):
            col = pl.multiple_of(col0 + ju * L, L)
            pe_reg = pe_v[s_loc, pl.ds(col, L)]
            for b in range(B):
              r = s_loc * B + b
              rows_v[r, pl.ds(col, L)] = rows_v[r, pl.ds(col, L)] * SCALE + pe_reg
          return carry2
        lax.fori_loop(0, (D // L) // UJ, j_body, 0)
        return carry
      lax.fori_loop(0, SR, s_body, 0)

    loads = [None] * NCH
    wbs = [None] * NCH
    loads[0] = start_load(0)
    for c in range(NCH):
      buf = c % 2
      row0 = pl.multiple_of(base + c * CH, CH)
      for g in loads[c]:
        g.wait()
      if c + 1 < NCH:
        if c >= 1:
          wbs[c - 1].wait()
        loads[c + 1] = start_load(c + 1)
      compute(rows_b[buf], pe_b[buf])
      wbs[c] = pltpu.async_copy(rows_b[buf], out2.at[pl.ds(row0, CH)], ws[buf])
    wbs[NCH - 2].wait()
    wbs[NCH - 1].wait()

  return k


def kernel(x, W, pe):
  S, B = x.shape
  V, D = W.shape
  x_flat = x.reshape(S * B)
  return _make_sc_kernel(V, D, S, B, pe.shape[0])(W, x_flat, pe)
